# Initial kernel scaffold; baseline (speedup 1.0000x reference)
import jax
import jax.numpy as jnp
from jax.experimental import pallas as pl


def kernel(query, key, value, Wq, bq, Wk, bk, Wv, bv, U, V, omega, rff_bias, lsh_vecs, Wo, bo):
    raise NotImplementedError("write your pallas kernel here")


# speedup vs baseline: 35.8609x; 35.8609x over previous
import functools
import math

import jax
import jax.numpy as jnp
from jax.experimental import pallas as pl
from jax.experimental.pallas import tpu as pltpu

L = 2048
D_MODEL = 768
H = 12
DQ = 64
DK = 64
RANK = 32
RFF = 64
KMAX = 16
BUCKETS = 4
BAND = 4.0
NH = 2
NB = BUCKETS ** NH
NSLOT = NB * KMAX
RB = 256
NRB = L // RB

_HI = jax.lax.Precision.HIGHEST
_DEF = jax.lax.Precision.DEFAULT


def _proj_hash_body(xq_ref, xk_ref, xv_ref, wq_ref, wk_ref, wv_ref,
                    bq_ref, bk_ref, bv_ref, lsh_ref, e12_ref,
                    q_ref, k_ref, v_ref, qoh_ref, koh_ref):
    q = jnp.dot(xq_ref[...], wq_ref[...], precision=_DEF) + bq_ref[...]
    k = jnp.dot(xk_ref[...], wk_ref[...], precision=_DEF) + bk_ref[...]
    v = jnp.dot(xv_ref[...], wv_ref[...], precision=_DEF) + bv_ref[...]
    q_ref[...] = q
    k_ref[...] = k
    v_ref[...] = v
    lsh = lsh_ref[...]
    for (x, oh_ref) in ((q, qoh_ref), (k, koh_ref)):
        xb = (x > 0).astype(jnp.float32)
        hv = jnp.dot(xb, lsh, precision=_DEF)
        hq = jnp.floor(hv / BAND) % BUCKETS
        comb = hq[:, :H] * BUCKETS + hq[:, H:]
        cexp = jnp.dot(comb, e12_ref[...], precision=_HI)
        ccol = (jax.lax.broadcasted_iota(jnp.int32, (1, H * NB), 1) % NB).astype(jnp.float32)
        oh_ref[...] = (cexp == ccol).astype(jnp.float32)


def _rank_body(koh_ref, asel_ref, cnt_ref, carry):
    i = pl.program_id(0)

    @pl.when(i == 0)
    def _():
        carry[...] = jnp.zeros_like(carry)

    oh = koh_ref[...]
    r_iota = jax.lax.broadcasted_iota(jnp.int32, (RB, RB), 0)
    c_iota = jax.lax.broadcasted_iota(jnp.int32, (RB, RB), 1)
    tri = (r_iota >= c_iota).astype(jnp.float32)
    cum = jnp.dot(tri, oh, precision=_HI) + carry[...]
    carry[...] = cum[RB - 1:RB, :]
    cnt_ref[...] = cum[RB - 1:RB, :]
    cumc = jnp.minimum(cum, 17.0)
    g = _rep16_pattern()
    cume = jnp.dot(cumc, g, precision=_HI)
    ohe = jnp.dot(oh, g, precision=_HI)
    tcol = (jax.lax.broadcasted_iota(jnp.int32, (1, H * NSLOT), 1) % KMAX).astype(jnp.float32)
    asel_ref[...] = ((cume == tcol + 1.0) & (ohe > 0.5)).astype(jnp.float32)


def _rep16_pattern():
    src = jax.lax.broadcasted_iota(jnp.int32, (H * NB, H * NSLOT), 0)
    dst = jax.lax.broadcasted_iota(jnp.int32, (H * NB, H * NSLOT), 1)
    return (src == dst // KMAX).astype(jnp.float32)


def _gather_body(asel_ref, k_ref, v_ref, ksel_ref, vsel_ref):
    a = asel_ref[...]
    dn = (((0,), (0,)), ((), ()))
    ksel_ref[0] = jax.lax.dot_general(a, k_ref[0], dn, precision=_HI)
    vsel_ref[0] = jax.lax.dot_general(a, v_ref[0], dn, precision=_HI)


def _pproj_body(v_ref, wo_ref, p_ref):
    p_ref[0] = jnp.dot(v_ref[0], wo_ref[...], precision=_HI)


def _attn_body(q_ref, ksel_ref, vsel_ref, om_ref, rb_ref, u_ref, p_ref,
               cnt_ref, qoh_ref, out_ref):
    h = pl.program_id(0)
    rff_scale = math.sqrt(2.0 / RFF)
    om = om_ref[0]
    rb = rb_ref[0]
    q_r = jnp.cos(jnp.dot(q_ref[0], om, precision=_HI) + rb) * rff_scale
    ks_r = jnp.cos(jnp.dot(ksel_ref[0], om, precision=_HI) + rb) * rff_scale
    dn = (((1,), (1,)), ((), ()))
    s = jax.lax.dot_general(q_r, ks_r, dn, precision=_HI) * (1.0 / math.sqrt(RFF))
    cnt = jnp.minimum(cnt_ref[0], 17.0)
    g16 = (jax.lax.broadcasted_iota(jnp.int32, (NB, NSLOT), 0)
           == jax.lax.broadcasted_iota(jnp.int32, (NB, NSLOT), 1) // KMAX
           ).astype(jnp.float32)
    cexp = jnp.dot(cnt, g16, precision=_HI)
    tcol = (jax.lax.broadcasted_iota(jnp.int32, (1, NSLOT), 1) % KMAX).astype(jnp.float32)
    occ = cexp > tcol
    qexp = jnp.dot(qoh_ref[0], g16, precision=_HI)
    mask = (qexp > 0.5) & occ
    s = jnp.where(mask, s, -jnp.inf)
    mx = jnp.max(s, axis=1, keepdims=True)
    e = jnp.exp(s - mx)
    attn = e / jnp.sum(e, axis=1, keepdims=True)
    wv = jnp.dot(attn, vsel_ref[0], precision=_HI)
    t = jnp.dot(wv, u_ref[0], precision=_HI)
    y = jnp.dot(t, p_ref[0], precision=_HI)

    @pl.when(h == 0)
    def _():
        out_ref[...] = y

    @pl.when(h > 0)
    def _():
        out_ref[...] = out_ref[...] + y


def kernel(query, key, value, Wq, bq, Wk, bk, Wv, bv, U, V, omega, rff_bias,
           lsh_vecs, Wo, bo):
    f32 = jnp.float32
    xq = query[0]
    xk = key[0]
    xv = value[0]
    lshbd = jnp.zeros((H, DQ, 2 * H), f32)
    idx = jnp.arange(H)
    lshbd = lshbd.at[idx, :, idx].set(lsh_vecs[:, :, 0])
    lshbd = lshbd.at[idx, :, idx + H].set(lsh_vecs[:, :, 1])
    lshbd = lshbd.reshape(H * DQ, 2 * H)
    e12 = (jnp.arange(H)[:, None] == (jnp.arange(H * NB)[None, :] // NB)
           ).astype(f32)

    grid_a = (NRB,)
    rbs = lambda i: (i, 0)
    full = lambda i: (0, 0)
    q2, k2, v2, qoh, koh = pl.pallas_call(
        _proj_hash_body,
        grid=grid_a,
        in_specs=[
            pl.BlockSpec((RB, D_MODEL), rbs),
            pl.BlockSpec((RB, D_MODEL), rbs),
            pl.BlockSpec((RB, D_MODEL), rbs),
            pl.BlockSpec((D_MODEL, D_MODEL), full),
            pl.BlockSpec((D_MODEL, D_MODEL), full),
            pl.BlockSpec((D_MODEL, D_MODEL), full),
            pl.BlockSpec((1, D_MODEL), full),
            pl.BlockSpec((1, D_MODEL), full),
            pl.BlockSpec((1, D_MODEL), full),
            pl.BlockSpec((D_MODEL, 2 * H), full),
            pl.BlockSpec((H, H * NB), full),
        ],
        out_specs=[
            pl.BlockSpec((RB, D_MODEL), rbs),
            pl.BlockSpec((RB, D_MODEL), rbs),
            pl.BlockSpec((RB, D_MODEL), rbs),
            pl.BlockSpec((RB, H * NB), rbs),
            pl.BlockSpec((RB, H * NB), rbs),
        ],
        out_shape=[
            jax.ShapeDtypeStruct((L, D_MODEL), f32),
            jax.ShapeDtypeStruct((L, D_MODEL), f32),
            jax.ShapeDtypeStruct((L, D_MODEL), f32),
            jax.ShapeDtypeStruct((L, H * NB), f32),
            jax.ShapeDtypeStruct((L, H * NB), f32),
        ],
    )(xq, xk, xv, Wq, Wk, Wv, bq[None], bk[None], bv[None], lshbd, e12)

    asel, cnt = pl.pallas_call(
        _rank_body,
        grid=(NRB,),
        in_specs=[pl.BlockSpec((RB, H * NB), rbs)],
        out_specs=[
            pl.BlockSpec((RB, H * NSLOT), rbs),
            pl.BlockSpec((1, H * NB), full),
        ],
        out_shape=[
            jax.ShapeDtypeStruct((L, H * NSLOT), f32),
            jax.ShapeDtypeStruct((1, H * NB), f32),
        ],
        scratch_shapes=[pltpu.VMEM((1, H * NB), f32)],
    )(koh)

    q3 = q2.reshape(L, H, DK).transpose(1, 0, 2)
    k3 = k2.reshape(L, H, DK).transpose(1, 0, 2)
    v3 = v2.reshape(L, H, DK).transpose(1, 0, 2)
    qoh3 = qoh.reshape(L, H, NB).transpose(1, 0, 2)
    cnt3 = cnt.reshape(H, NB)[:, None, :]

    ksel, vsel = pl.pallas_call(
        _gather_body,
        grid=(H,),
        in_specs=[
            pl.BlockSpec((L, NSLOT), lambda h: (0, h)),
            pl.BlockSpec((1, L, DK), lambda h: (h, 0, 0)),
            pl.BlockSpec((1, L, DK), lambda h: (h, 0, 0)),
        ],
        out_specs=[
            pl.BlockSpec((1, NSLOT, DK), lambda h: (h, 0, 0)),
            pl.BlockSpec((1, NSLOT, DK), lambda h: (h, 0, 0)),
        ],
        out_shape=[
            jax.ShapeDtypeStruct((H, NSLOT, DK), f32),
            jax.ShapeDtypeStruct((H, NSLOT, DK), f32),
        ],
    )(asel, k3, v3)

    p = pl.pallas_call(
        _pproj_body,
        grid=(H,),
        in_specs=[
            pl.BlockSpec((1, RANK, D_MODEL), lambda h: (h, 0, 0)),
            pl.BlockSpec((D_MODEL, D_MODEL), lambda h: (h, 0)),
        ],
        out_specs=pl.BlockSpec((1, RANK, D_MODEL), lambda h: (h, 0, 0)),
        out_shape=jax.ShapeDtypeStruct((H, RANK, D_MODEL), f32),
    )(V, Wo)

    out = pl.pallas_call(
        _attn_body,
        grid=(H,),
        in_specs=[
            pl.BlockSpec((1, L, DK), lambda h: (h, 0, 0)),
            pl.BlockSpec((1, NSLOT, DK), lambda h: (h, 0, 0)),
            pl.BlockSpec((1, NSLOT, DK), lambda h: (h, 0, 0)),
            pl.BlockSpec((1, DK, RFF), lambda h: (h, 0, 0)),
            pl.BlockSpec((1, 1, RFF), lambda h: (h, 0, 0)),
            pl.BlockSpec((1, DK, RANK), lambda h: (h, 0, 0)),
            pl.BlockSpec((1, RANK, D_MODEL), lambda h: (h, 0, 0)),
            pl.BlockSpec((1, 1, NB), lambda h: (h, 0, 0)),
            pl.BlockSpec((1, L, NB), lambda h: (h, 0, 0)),
        ],
        out_specs=pl.BlockSpec((L, D_MODEL), lambda h: (0, 0)),
        out_shape=jax.ShapeDtypeStruct((L, D_MODEL), f32),
    )(q3, ksel, vsel, omega, rff_bias[:, None, :], U, p, cnt3, qoh3)

    return (out + bo)[None]


# speedup vs baseline: 37.5025x; 1.0458x over previous
import functools
import math

import jax
import jax.numpy as jnp
from jax import lax
from jax.experimental import pallas as pl
from jax.experimental.pallas import tpu as pltpu
from jax.experimental.pallas import tpu_sc as plsc

L = 2048
D_MODEL = 768
H = 12
DQ = 64
DK = 64
RANK = 32
RFF = 64
KMAX = 16
BUCKETS = 4
BAND = 4.0
NH = 2
NB = BUCKETS ** NH
NSLOT = NB * KMAX
NPAIR = H * NB
NROW = H * NSLOT
RB = 256
NRB = L // RB

_HI = jax.lax.Precision.HIGHEST
_DEF = jax.lax.Precision.DEFAULT


def _proj_hash_body(xq_ref, xk_ref, xv_ref, wq_ref, wk_ref, wv_ref,
                    bq_ref, bk_ref, bv_ref, lsh_ref, e12_ref,
                    q_ref, k_ref, v_ref, qoh_ref, koh_ref):
    q = jnp.dot(xq_ref[...], wq_ref[...], precision=_DEF) + bq_ref[...]
    k = jnp.dot(xk_ref[...], wk_ref[...], precision=_DEF) + bk_ref[...]
    v = jnp.dot(xv_ref[...], wv_ref[...], precision=_DEF) + bv_ref[...]
    q_ref[...] = q
    k_ref[...] = k
    v_ref[...] = v
    lsh = lsh_ref[...]
    ccol = (jax.lax.broadcasted_iota(jnp.int32, (1, H * NB), 1) % NB
            ).astype(jnp.float32)
    for (x, oh_ref) in ((q, qoh_ref), (k, koh_ref)):
        xb = (x > 0).astype(jnp.float32)
        hv = jnp.dot(xb, lsh, precision=_DEF)
        hq = jnp.floor(hv / BAND) % BUCKETS
        comb = hq[:, :H] * BUCKETS + hq[:, H:]
        cexp = jnp.dot(comb, e12_ref[...], precision=_HI)
        oh_ref[...] = (cexp == ccol).astype(jnp.float32)


def _rep16_pattern():
    src = jax.lax.broadcasted_iota(jnp.int32, (H * NB, NROW), 0)
    dst = jax.lax.broadcasted_iota(jnp.int32, (H * NB, NROW), 1)
    return (src == dst // KMAX).astype(jnp.float32)


def _rank_body(koh_ref, cnt_ref, tbl_ref, carry, tblacc):
    i = pl.program_id(0)

    @pl.when(i == 0)
    def _():
        carry[...] = jnp.zeros_like(carry)
        tblacc[...] = jnp.zeros_like(tblacc)

    oh = koh_ref[...]
    r_iota = jax.lax.broadcasted_iota(jnp.int32, (RB, RB), 0)
    c_iota = jax.lax.broadcasted_iota(jnp.int32, (RB, RB), 1)
    tri = (r_iota >= c_iota).astype(jnp.float32)
    cum = jnp.dot(tri, oh, precision=_HI) + carry[...]
    carry[...] = cum[RB - 1:RB, :]
    cnt_ref[...] = cum[RB - 1:RB, :]
    cumc = jnp.minimum(cum, 17.0)
    g = _rep16_pattern()
    cume = jnp.dot(cumc, g, precision=_HI)
    ohe = jnp.dot(oh, g, precision=_HI)
    tcol = (jax.lax.broadcasted_iota(jnp.int32, (1, NROW), 1) % KMAX
            ).astype(jnp.float32)
    asel = ((cume == tcol + 1.0) & (ohe > 0.5)).astype(jnp.float32)
    jrow = (jax.lax.broadcasted_iota(jnp.int32, (1, RB), 1) + i * RB
            ).astype(jnp.float32)
    tblacc[...] = tblacc[...] + jnp.dot(jrow, asel, precision=_HI)
    hofs = ((jax.lax.broadcasted_iota(jnp.int32, (1, NROW), 1) // NSLOT) * L
            ).astype(jnp.float32)
    tbl_ref[...] = tblacc[...] + hofs


def _sc_gather(tbl_hbm, kvflat_hbm, kvsel_hbm, tblv, rows_kv, sem):
    info = plsc.get_sparse_core_info()
    nw = info.num_cores * info.num_subcores
    wid = lax.axis_index("s") * info.num_cores + lax.axis_index("c")
    nrow_per = NROW // nw
    base = wid * nrow_per
    pltpu.sync_copy(tbl_hbm.at[pl.ds(base, nrow_per)], tblv)
    pltpu.async_copy(kvflat_hbm.at[tblv], rows_kv, sem).wait()
    pltpu.sync_copy(rows_kv, kvsel_hbm.at[pl.ds(base, nrow_per)])


def _pproj_body(v_ref, wo_ref, p_ref):
    p_ref[0] = jnp.dot(v_ref[0], wo_ref[...], precision=_HI)


def _attn_body(q_ref, ksel_ref, vsel_ref, om_ref, rb_ref, u_ref, p_ref,
               cnt_ref, qoh_ref, out_ref):
    h = pl.program_id(0)
    rff_scale = math.sqrt(2.0 / RFF)
    om = om_ref[0]
    rb = rb_ref[0]
    q_r = jnp.cos(jnp.dot(q_ref[0], om, precision=_HI) + rb) * rff_scale
    ks_r = jnp.cos(jnp.dot(ksel_ref[0], om, precision=_HI) + rb) * rff_scale
    dn = (((1,), (1,)), ((), ()))
    s = jax.lax.dot_general(q_r, ks_r, dn, precision=_HI) * (1.0 / math.sqrt(RFF))
    cnt = jnp.minimum(cnt_ref[0], 17.0)
    g16 = (jax.lax.broadcasted_iota(jnp.int32, (NB, NSLOT), 0)
           == jax.lax.broadcasted_iota(jnp.int32, (NB, NSLOT), 1) // KMAX
           ).astype(jnp.float32)
    cexp = jnp.dot(cnt, g16, precision=_HI)
    tcol = (jax.lax.broadcasted_iota(jnp.int32, (1, NSLOT), 1) % KMAX
            ).astype(jnp.float32)
    occ = cexp > tcol
    qexp = jnp.dot(qoh_ref[0], g16, precision=_HI)
    mask = (qexp > 0.5) & occ
    s = jnp.where(mask, s, -jnp.inf)
    mx = jnp.max(s, axis=1, keepdims=True)
    e = jnp.exp(s - mx)
    attn = e / jnp.sum(e, axis=1, keepdims=True)
    wv = jnp.dot(attn, vsel_ref[0], precision=_HI)
    t = jnp.dot(wv, u_ref[0], precision=_HI)
    y = jnp.dot(t, p_ref[0], precision=_HI)

    @pl.when(h == 0)
    def _():
        out_ref[...] = y

    @pl.when(h > 0)
    def _():
        out_ref[...] = out_ref[...] + y


def kernel(query, key, value, Wq, bq, Wk, bk, Wv, bv, U, V, omega, rff_bias,
           lsh_vecs, Wo, bo):
    f32 = jnp.float32
    xq = query[0]
    xk = key[0]
    xv = value[0]
    lshbd = jnp.zeros((H, DQ, 2 * H), f32)
    idx = jnp.arange(H)
    lshbd = lshbd.at[idx, :, idx].set(lsh_vecs[:, :, 0])
    lshbd = lshbd.at[idx, :, idx + H].set(lsh_vecs[:, :, 1])
    lshbd = lshbd.reshape(H * DQ, 2 * H)
    e12 = (jnp.arange(H)[:, None] == (jnp.arange(H * NB)[None, :] // NB)
           ).astype(f32)

    rbs = lambda i: (i, 0)
    full = lambda i: (0, 0)
    q2, k2, v2, qoh, koh = pl.pallas_call(
        _proj_hash_body,
        grid=(NRB,),
        in_specs=[
            pl.BlockSpec((RB, D_MODEL), rbs),
            pl.BlockSpec((RB, D_MODEL), rbs),
            pl.BlockSpec((RB, D_MODEL), rbs),
            pl.BlockSpec((D_MODEL, D_MODEL), full),
            pl.BlockSpec((D_MODEL, D_MODEL), full),
            pl.BlockSpec((D_MODEL, D_MODEL), full),
            pl.BlockSpec((1, D_MODEL), full),
            pl.BlockSpec((1, D_MODEL), full),
            pl.BlockSpec((1, D_MODEL), full),
            pl.BlockSpec((D_MODEL, 2 * H), full),
            pl.BlockSpec((H, H * NB), full),
        ],
        out_specs=[
            pl.BlockSpec((RB, D_MODEL), rbs),
            pl.BlockSpec((RB, D_MODEL), rbs),
            pl.BlockSpec((RB, D_MODEL), rbs),
            pl.BlockSpec((RB, H * NB), rbs),
            pl.BlockSpec((RB, H * NB), rbs),
        ],
        out_shape=[
            jax.ShapeDtypeStruct((L, D_MODEL), f32),
            jax.ShapeDtypeStruct((L, D_MODEL), f32),
            jax.ShapeDtypeStruct((L, D_MODEL), f32),
            jax.ShapeDtypeStruct((L, H * NB), f32),
            jax.ShapeDtypeStruct((L, H * NB), f32),
        ],
    )(xq, xk, xv, Wq, Wk, Wv, bq[None], bk[None], bv[None], lshbd, e12)

    cnt, tbl = pl.pallas_call(
        _rank_body,
        grid=(NRB,),
        in_specs=[pl.BlockSpec((RB, H * NB), rbs)],
        out_specs=[
            pl.BlockSpec((1, H * NB), full),
            pl.BlockSpec((1, NROW), full),
        ],
        out_shape=[
            jax.ShapeDtypeStruct((1, H * NB), f32),
            jax.ShapeDtypeStruct((1, NROW), f32),
        ],
        scratch_shapes=[
            pltpu.VMEM((1, H * NB), f32),
            pltpu.VMEM((1, NROW), f32),
        ],
    )(koh)

    q3 = q2.reshape(L, H, DK).transpose(1, 0, 2)
    k3 = k2.reshape(L, H, DK).transpose(1, 0, 2)
    v3 = v2.reshape(L, H, DK).transpose(1, 0, 2)
    qoh3 = qoh.reshape(L, H, NB).transpose(1, 0, 2)
    cnt3 = cnt.reshape(H, NB)[:, None, :]
    tbl_i = tbl.reshape(NROW).astype(jnp.int32)

    kvflat = jnp.concatenate([k3, v3], axis=-1).reshape(H * L, 2 * DK)

    mesh = plsc.VectorSubcoreMesh(core_axis_name="c", subcore_axis_name="s")
    kvsel = functools.partial(
        pl.kernel,
        mesh=mesh,
        out_type=jax.ShapeDtypeStruct((NROW, 2 * DK), f32),
        scratch_types=[
            pltpu.VMEM((NROW // 32,), jnp.int32),
            pltpu.VMEM((NROW // 32, 2 * DK), f32),
            pltpu.SemaphoreType.DMA,
        ],
    )(_sc_gather)(tbl_i, kvflat)

    ksel = kvsel[:, :DK].reshape(H, NSLOT, DK)
    vsel = kvsel[:, DK:].reshape(H, NSLOT, DK)

    p = pl.pallas_call(
        _pproj_body,
        grid=(H,),
        in_specs=[
            pl.BlockSpec((1, RANK, D_MODEL), lambda h: (h, 0, 0)),
            pl.BlockSpec((D_MODEL, D_MODEL), lambda h: (h, 0)),
        ],
        out_specs=pl.BlockSpec((1, RANK, D_MODEL), lambda h: (h, 0, 0)),
        out_shape=jax.ShapeDtypeStruct((H, RANK, D_MODEL), f32),
    )(V, Wo)

    out = pl.pallas_call(
        _attn_body,
        grid=(H,),
        in_specs=[
            pl.BlockSpec((1, L, DK), lambda h: (h, 0, 0)),
            pl.BlockSpec((1, NSLOT, DK), lambda h: (h, 0, 0)),
            pl.BlockSpec((1, NSLOT, DK), lambda h: (h, 0, 0)),
            pl.BlockSpec((1, DK, RFF), lambda h: (h, 0, 0)),
            pl.BlockSpec((1, 1, RFF), lambda h: (h, 0, 0)),
            pl.BlockSpec((1, DK, RANK), lambda h: (h, 0, 0)),
            pl.BlockSpec((1, RANK, D_MODEL), lambda h: (h, 0, 0)),
            pl.BlockSpec((1, 1, NB), lambda h: (h, 0, 0)),
            pl.BlockSpec((1, L, NB), lambda h: (h, 0, 0)),
        ],
        out_specs=pl.BlockSpec((L, D_MODEL), lambda h: (0, 0)),
        out_shape=jax.ShapeDtypeStruct((L, D_MODEL), f32),
    )(q3, ksel, vsel, omega, rff_bias[:, None, :], U, p, cnt3, qoh3)

    return (out + bo)[None]


# speedup vs baseline: 62.7203x; 1.6724x over previous
import functools
import math

import jax
import jax.numpy as jnp
from jax import lax
from jax.experimental import pallas as pl
from jax.experimental.pallas import tpu as pltpu
from jax.experimental.pallas import tpu_sc as plsc

L = 2048
D_MODEL = 768
H = 12
DQ = 64
DK = 64
RANK = 32
RFF = 64
KMAX = 16
BUCKETS = 4
BAND = 4.0
NH = 2
NB = BUCKETS ** NH
NSLOT = NB * KMAX
NROW = H * NSLOT
RB = 256
NRB = L // RB

_HI = jax.lax.Precision.HIGHEST
_DEF = jax.lax.Precision.DEFAULT


def _rep16_pattern():
    src = jax.lax.broadcasted_iota(jnp.int32, (H * NB, NROW), 0)
    dst = jax.lax.broadcasted_iota(jnp.int32, (H * NB, NROW), 1)
    return (src == dst // KMAX).astype(jnp.float32)


def _projrank_body(xq_ref, xk_ref, xv_ref, wq_ref, wk_ref, wv_ref,
                   bq_ref, bk_ref, bv_ref, lsh_ref, e12_ref,
                   q_ref, kv_ref, v_ref, qoh_ref, cnt_ref, tbl_ref,
                   carry, tblacc):
    i = pl.program_id(0)

    @pl.when(i == 0)
    def _():
        carry[...] = jnp.zeros_like(carry)
        tblacc[...] = jnp.zeros_like(tblacc)

    q = jnp.dot(xq_ref[...], wq_ref[...], precision=_DEF) + bq_ref[...]
    k = jnp.dot(xk_ref[...], wk_ref[...], precision=_DEF) + bk_ref[...]
    v = jnp.dot(xv_ref[...], wv_ref[...], precision=_DEF) + bv_ref[...]
    q_ref[...] = q
    v_ref[...] = v
    pieces = []
    for h in range(H):
        pieces.append(k[:, h * DK:(h + 1) * DK])
        pieces.append(v[:, h * DK:(h + 1) * DK])
    kv_ref[...] = jnp.concatenate(pieces, axis=1)
    lsh = lsh_ref[...]
    ccol = (jax.lax.broadcasted_iota(jnp.int32, (1, H * NB), 1) % NB
            ).astype(jnp.float32)
    ohs = []
    for x in (q, k):
        xb = (x > 0).astype(jnp.float32)
        hv = jnp.dot(xb, lsh, precision=_DEF)
        hq = jnp.floor(hv / BAND) % BUCKETS
        comb = hq[:, :H] * BUCKETS + hq[:, H:]
        cexp = jnp.dot(comb, e12_ref[...], precision=_DEF)
        ohs.append((cexp == ccol).astype(jnp.float32))
    qoh_ref[...] = ohs[0]
    oh = ohs[1]
    r_iota = jax.lax.broadcasted_iota(jnp.int32, (RB, RB), 0)
    c_iota = jax.lax.broadcasted_iota(jnp.int32, (RB, RB), 1)
    tri = (r_iota >= c_iota).astype(jnp.float32)
    cum = jnp.dot(tri, oh, precision=_DEF) + carry[...]
    carry[...] = cum[RB - 1:RB, :]
    cnt_ref[...] = cum[RB - 1:RB, :]
    cumc = jnp.minimum(cum, 17.0)
    g = _rep16_pattern()
    cume = jnp.dot(cumc, g, precision=_DEF)
    ohe = jnp.dot(oh, g, precision=_DEF)
    tcol = (jax.lax.broadcasted_iota(jnp.int32, (1, NROW), 1) % KMAX
            ).astype(jnp.float32)
    asel = ((cume == tcol + 1.0) & (ohe > 0.5)).astype(jnp.float32)
    jrow = (jax.lax.broadcasted_iota(jnp.int32, (1, RB), 1) + i * RB
            ).astype(jnp.float32)
    tblacc[...] = tblacc[...] + jnp.dot(jrow, asel, precision=_HI)
    hcol = (jax.lax.broadcasted_iota(jnp.int32, (1, NROW), 1) // NSLOT
            ).astype(jnp.float32)
    tbl_ref[...] = tblacc[...] * H + hcol


def _sc_gather(tbl_hbm, kvflat_hbm, kvsel_hbm, tblv, rows_kv, sem):
    info = plsc.get_sparse_core_info()
    nw = info.num_cores * info.num_subcores
    wid = lax.axis_index("s") * info.num_cores + lax.axis_index("c")
    nrow_per = NROW // nw
    base = wid * nrow_per
    pltpu.sync_copy(tbl_hbm.at[pl.ds(base, nrow_per)], tblv)
    pltpu.async_copy(kvflat_hbm.at[tblv], rows_kv, sem).wait()
    pltpu.sync_copy(rows_kv, kvsel_hbm.at[pl.ds(base, nrow_per)])


def _attn_body(q_ref, kvsel_ref, om_ref, rb_ref, u_ref,
               vv_ref, wo_ref, cnt_ref, qoh_ref, out_ref):
    h = pl.program_id(0)
    rff_scale = math.sqrt(2.0 / RFF)
    om = om_ref[0]
    rb = rb_ref[0]
    kv = kvsel_ref[0]
    ksel = kv[:, :DK]
    vsel = kv[:, DK:]
    q_r = jnp.cos(jnp.dot(q_ref[0], om, precision=_DEF) + rb) * rff_scale
    ks_r = jnp.cos(jnp.dot(ksel, om, precision=_DEF) + rb) * rff_scale
    dn = (((1,), (1,)), ((), ()))
    s = jax.lax.dot_general(q_r, ks_r, dn, precision=_DEF) * (1.0 / math.sqrt(RFF))
    cnt = jnp.minimum(cnt_ref[0], 17.0)
    g16 = (jax.lax.broadcasted_iota(jnp.int32, (NB, NSLOT), 0)
           == jax.lax.broadcasted_iota(jnp.int32, (NB, NSLOT), 1) // KMAX
           ).astype(jnp.float32)
    cexp = jnp.dot(cnt, g16, precision=_DEF)
    tcol = (jax.lax.broadcasted_iota(jnp.int32, (1, NSLOT), 1) % KMAX
            ).astype(jnp.float32)
    occ = cexp > tcol
    qexp = jnp.dot(qoh_ref[0], g16, precision=_DEF)
    mask = (qexp > 0.5) & occ
    s = jnp.where(mask, s, -jnp.inf)
    mx = jnp.max(s, axis=1, keepdims=True)
    e = jnp.exp(s - mx)
    attn = e / jnp.sum(e, axis=1, keepdims=True)
    wv = jnp.dot(attn, vsel, precision=_DEF)
    t = jnp.dot(wv, u_ref[0], precision=_HI)
    p = jnp.dot(vv_ref[0], wo_ref[...], precision=_HI)
    y = jnp.dot(t, p, precision=_HI)

    @pl.when(h == 0)
    def _():
        out_ref[...] = y

    @pl.when(h > 0)
    def _():
        out_ref[...] = out_ref[...] + y


def kernel(query, key, value, Wq, bq, Wk, bk, Wv, bv, U, V, omega, rff_bias,
           lsh_vecs, Wo, bo):
    f32 = jnp.float32
    xq = query[0]
    xk = key[0]
    xv = value[0]
    lshbd = jnp.zeros((H, DQ, 2 * H), f32)
    idx = jnp.arange(H)
    lshbd = lshbd.at[idx, :, idx].set(lsh_vecs[:, :, 0])
    lshbd = lshbd.at[idx, :, idx + H].set(lsh_vecs[:, :, 1])
    lshbd = lshbd.reshape(H * DQ, 2 * H)
    e12 = (jnp.arange(H)[:, None] == (jnp.arange(H * NB)[None, :] // NB)
           ).astype(f32)

    rbs = lambda i: (i, 0)
    full = lambda i: (0, 0)
    q2, kv2, v2, qoh, cnt, tbl = pl.pallas_call(
        _projrank_body,
        grid=(NRB,),
        in_specs=[
            pl.BlockSpec((RB, D_MODEL), rbs),
            pl.BlockSpec((RB, D_MODEL), rbs),
            pl.BlockSpec((RB, D_MODEL), rbs),
            pl.BlockSpec((D_MODEL, D_MODEL), full),
            pl.BlockSpec((D_MODEL, D_MODEL), full),
            pl.BlockSpec((D_MODEL, D_MODEL), full),
            pl.BlockSpec((1, D_MODEL), full),
            pl.BlockSpec((1, D_MODEL), full),
            pl.BlockSpec((1, D_MODEL), full),
            pl.BlockSpec((D_MODEL, 2 * H), full),
            pl.BlockSpec((H, H * NB), full),
        ],
        out_specs=[
            pl.BlockSpec((RB, D_MODEL), rbs),
            pl.BlockSpec((RB, 2 * H * DK), rbs),
            pl.BlockSpec((RB, D_MODEL), rbs),
            pl.BlockSpec((RB, H * NB), rbs),
            pl.BlockSpec((1, H * NB), full),
            pl.BlockSpec((1, NROW), full),
        ],
        out_shape=[
            jax.ShapeDtypeStruct((L, D_MODEL), f32),
            jax.ShapeDtypeStruct((L, 2 * H * DK), f32),
            jax.ShapeDtypeStruct((L, D_MODEL), f32),
            jax.ShapeDtypeStruct((L, H * NB), f32),
            jax.ShapeDtypeStruct((1, H * NB), f32),
            jax.ShapeDtypeStruct((1, NROW), f32),
        ],
        scratch_shapes=[
            pltpu.VMEM((1, H * NB), f32),
            pltpu.VMEM((1, NROW), f32),
        ],
    )(xq, xk, xv, Wq, Wk, Wv, bq[None], bk[None], bv[None], lshbd, e12)

    q3 = q2.reshape(L, H, DK).transpose(1, 0, 2)
    qoh3 = qoh.reshape(L, H, NB).transpose(1, 0, 2)
    cnt3 = cnt.reshape(H, NB)[:, None, :]
    tbl_i = tbl.reshape(NROW).astype(jnp.int32)
    kvflat = kv2.reshape(L * H, 2 * DK)

    mesh = plsc.VectorSubcoreMesh(core_axis_name="c", subcore_axis_name="s")
    kvsel = functools.partial(
        pl.kernel,
        mesh=mesh,
        out_type=jax.ShapeDtypeStruct((NROW, 2 * DK), f32),
        scratch_types=[
            pltpu.VMEM((NROW // 32,), jnp.int32),
            pltpu.VMEM((NROW // 32, 2 * DK), f32),
            pltpu.SemaphoreType.DMA,
        ],
    )(_sc_gather)(tbl_i, kvflat)

    kvsel3 = kvsel.reshape(H, NSLOT, 2 * DK)

    out = pl.pallas_call(
        _attn_body,
        grid=(H,),
        in_specs=[
            pl.BlockSpec((1, L, DK), lambda h: (h, 0, 0)),
            pl.BlockSpec((1, NSLOT, 2 * DK), lambda h: (h, 0, 0)),
            pl.BlockSpec((1, DK, RFF), lambda h: (h, 0, 0)),
            pl.BlockSpec((1, 1, RFF), lambda h: (h, 0, 0)),
            pl.BlockSpec((1, DK, RANK), lambda h: (h, 0, 0)),
            pl.BlockSpec((1, RANK, D_MODEL), lambda h: (h, 0, 0)),
            pl.BlockSpec((D_MODEL, D_MODEL), lambda h: (h, 0)),
            pl.BlockSpec((1, 1, NB), lambda h: (h, 0, 0)),
            pl.BlockSpec((1, L, NB), lambda h: (h, 0, 0)),
        ],
        out_specs=pl.BlockSpec((L, D_MODEL), lambda h: (0, 0)),
        out_shape=jax.ShapeDtypeStruct((L, D_MODEL), f32),
    )(q3, kvsel3, omega, rff_bias[:, None, :], U, V, Wo, cnt3, qoh3)

    return (out + bo)[None]


# speedup vs baseline: 84.3732x; 1.3452x over previous
import functools
import math

import jax
import jax.numpy as jnp
from jax import lax
from jax.experimental import pallas as pl
from jax.experimental.pallas import tpu as pltpu
from jax.experimental.pallas import tpu_sc as plsc

L = 2048
D_MODEL = 768
H = 12
DQ = 64
DK = 64
RANK = 32
RFF = 64
KMAX = 16
BUCKETS = 4
BAND = 4.0
NH = 2
NB = BUCKETS ** NH
NSLOT = NB * KMAX
NROW = H * NSLOT
RB = 256
NRB = L // RB

_HI = jax.lax.Precision.HIGHEST
_DEF = jax.lax.Precision.DEFAULT


def _rep16_pattern():
    src = jax.lax.broadcasted_iota(jnp.int32, (H * NB, NROW), 0)
    dst = jax.lax.broadcasted_iota(jnp.int32, (H * NB, NROW), 1)
    return (src == dst // KMAX).astype(jnp.float32)


def _projrank_body(xq_ref, xk_ref, xv_ref, wq_ref, wk_ref, wv_ref,
                   bq_ref, bk_ref, bv_ref, lsh_ref, e12_ref,
                   q_ref, kv_ref, v_ref, qoh_ref, cnt_ref, tbl_ref,
                   carry, tblacc):
    i = pl.program_id(0)

    @pl.when(i == 0)
    def _():
        carry[...] = jnp.zeros_like(carry)
        tblacc[...] = jnp.zeros_like(tblacc)

    q = jnp.dot(xq_ref[...], wq_ref[...], precision=_DEF) + bq_ref[...]
    k = jnp.dot(xk_ref[...], wk_ref[...], precision=_DEF) + bk_ref[...]
    v = jnp.dot(xv_ref[...], wv_ref[...], precision=_DEF) + bv_ref[...]
    q_ref[...] = q
    v_ref[...] = v
    pieces = []
    for h in range(H):
        pieces.append(k[:, h * DK:(h + 1) * DK])
        pieces.append(v[:, h * DK:(h + 1) * DK])
    kv_ref[...] = jnp.concatenate(pieces, axis=1)
    lsh = lsh_ref[...]
    ccol = (jax.lax.broadcasted_iota(jnp.int32, (1, H * NB), 1) % NB
            ).astype(jnp.float32)
    ohs = []
    for x in (q, k):
        xb = (x > 0).astype(jnp.float32)
        hv = jnp.dot(xb, lsh, precision=_DEF)
        hq = jnp.floor(hv / BAND) % BUCKETS
        comb = hq[:, :H] * BUCKETS + hq[:, H:]
        cexp = jnp.dot(comb, e12_ref[...], precision=_DEF)
        ohs.append((cexp == ccol).astype(jnp.float32))
    qoh_ref[...] = ohs[0]
    oh = ohs[1]
    r_iota = jax.lax.broadcasted_iota(jnp.int32, (RB, RB), 0)
    c_iota = jax.lax.broadcasted_iota(jnp.int32, (RB, RB), 1)
    tri = (r_iota >= c_iota).astype(jnp.float32)
    cum = jnp.dot(tri, oh, precision=_DEF) + carry[...]
    carry[...] = cum[RB - 1:RB, :]
    cnt_ref[...] = cum[RB - 1:RB, :]
    cumc = jnp.minimum(cum, 17.0)
    g = _rep16_pattern()
    cume = jnp.dot(cumc, g, precision=_DEF)
    ohe = jnp.dot(oh, g, precision=_DEF)
    tcol = (jax.lax.broadcasted_iota(jnp.int32, (1, NROW), 1) % KMAX
            ).astype(jnp.float32)
    asel = ((cume == tcol + 1.0) & (ohe > 0.5)).astype(jnp.float32)
    jrow = (jax.lax.broadcasted_iota(jnp.int32, (1, RB), 1) + i * RB
            ).astype(jnp.float32)
    tblacc[...] = tblacc[...] + jnp.dot(jrow, asel, precision=_HI)
    hcol = (jax.lax.broadcasted_iota(jnp.int32, (1, NROW), 1) // NSLOT
            ).astype(jnp.float32)
    tbl_ref[...] = tblacc[...] * H + hcol


def _sc_gather(tbl_hbm, kvflat_hbm, kvsel_hbm, tblv, rows_kv, sem):
    info = plsc.get_sparse_core_info()
    nw = info.num_cores * info.num_subcores
    wid = lax.axis_index("s") * info.num_cores + lax.axis_index("c")
    nrow_per = NROW // nw
    base = wid * nrow_per
    pltpu.sync_copy(tbl_hbm.at[pl.ds(base, nrow_per)], tblv)
    pltpu.async_copy(kvflat_hbm.at[tblv], rows_kv, sem).wait()
    pltpu.sync_copy(rows_kv, kvsel_hbm.at[pl.ds(base, nrow_per)])


def _attn_body(q_ref, kvsel_ref, om_ref, rb_ref, u_ref,
               vv_ref, wo_ref, cnt_ref, qoh_ref, out_ref):
    h = pl.program_id(0)
    rff_scale = math.sqrt(2.0 / RFF)
    om = om_ref[0]
    rb = rb_ref[0]
    kv = kvsel_ref[0]
    ksel = kv[:, :DK]
    vsel = kv[:, DK:]
    q_r = jnp.cos(jnp.dot(q_ref[0], om, precision=_DEF) + rb) * rff_scale
    ks_r = jnp.cos(jnp.dot(ksel, om, precision=_DEF) + rb) * rff_scale
    dn = (((1,), (1,)), ((), ()))
    s = jax.lax.dot_general(q_r, ks_r, dn, precision=_DEF) * (1.0 / math.sqrt(RFF))
    cnt = jnp.minimum(cnt_ref[0], 17.0)
    g16 = (jax.lax.broadcasted_iota(jnp.int32, (NB, NSLOT), 0)
           == jax.lax.broadcasted_iota(jnp.int32, (NB, NSLOT), 1) // KMAX
           ).astype(jnp.float32)
    cexp = jnp.dot(cnt, g16, precision=_DEF)
    tcol = (jax.lax.broadcasted_iota(jnp.int32, (1, NSLOT), 1) % KMAX
            ).astype(jnp.float32)
    BIG = 1e30
    obias = jnp.where(cexp > tcol, -BIG, -2.0 * BIG)
    qexp = jnp.dot(qoh_ref[0], g16, precision=_DEF)
    s = s + (qexp * BIG + obias)
    mx = jnp.max(s, axis=1, keepdims=True)
    e = jnp.exp(s - mx)
    attn = e / jnp.sum(e, axis=1, keepdims=True)
    wv = jnp.dot(attn, vsel, precision=_DEF)
    t = jnp.dot(wv, u_ref[0], precision=_DEF)
    p = jnp.dot(vv_ref[0], wo_ref[...], precision=_DEF)
    y = jnp.dot(t, p, precision=_DEF)

    @pl.when(h == 0)
    def _():
        out_ref[...] = y

    @pl.when(h > 0)
    def _():
        out_ref[...] = out_ref[...] + y


def kernel(query, key, value, Wq, bq, Wk, bk, Wv, bv, U, V, omega, rff_bias,
           lsh_vecs, Wo, bo):
    f32 = jnp.float32
    xq = query[0]
    xk = key[0]
    xv = value[0]
    lshbd = jnp.zeros((H, DQ, 2 * H), f32)
    idx = jnp.arange(H)
    lshbd = lshbd.at[idx, :, idx].set(lsh_vecs[:, :, 0])
    lshbd = lshbd.at[idx, :, idx + H].set(lsh_vecs[:, :, 1])
    lshbd = lshbd.reshape(H * DQ, 2 * H)
    e12 = (jnp.arange(H)[:, None] == (jnp.arange(H * NB)[None, :] // NB)
           ).astype(f32)

    rbs = lambda i: (i, 0)
    full = lambda i: (0, 0)
    q2, kv2, v2, qoh, cnt, tbl = pl.pallas_call(
        _projrank_body,
        grid=(NRB,),
        in_specs=[
            pl.BlockSpec((RB, D_MODEL), rbs),
            pl.BlockSpec((RB, D_MODEL), rbs),
            pl.BlockSpec((RB, D_MODEL), rbs),
            pl.BlockSpec((D_MODEL, D_MODEL), full),
            pl.BlockSpec((D_MODEL, D_MODEL), full),
            pl.BlockSpec((D_MODEL, D_MODEL), full),
            pl.BlockSpec((1, D_MODEL), full),
            pl.BlockSpec((1, D_MODEL), full),
            pl.BlockSpec((1, D_MODEL), full),
            pl.BlockSpec((D_MODEL, 2 * H), full),
            pl.BlockSpec((H, H * NB), full),
        ],
        out_specs=[
            pl.BlockSpec((RB, D_MODEL), rbs),
            pl.BlockSpec((RB, 2 * H * DK), rbs),
            pl.BlockSpec((RB, D_MODEL), rbs),
            pl.BlockSpec((RB, H * NB), rbs),
            pl.BlockSpec((1, H * NB), full),
            pl.BlockSpec((1, NROW), full),
        ],
        out_shape=[
            jax.ShapeDtypeStruct((L, D_MODEL), f32),
            jax.ShapeDtypeStruct((L, 2 * H * DK), f32),
            jax.ShapeDtypeStruct((L, D_MODEL), f32),
            jax.ShapeDtypeStruct((L, H * NB), f32),
            jax.ShapeDtypeStruct((1, H * NB), f32),
            jax.ShapeDtypeStruct((1, NROW), f32),
        ],
        scratch_shapes=[
            pltpu.VMEM((1, H * NB), f32),
            pltpu.VMEM((1, NROW), f32),
        ],
    )(xq, xk, xv, Wq, Wk, Wv, bq[None], bk[None], bv[None], lshbd, e12)

    q3 = q2.reshape(L, H, DK).transpose(1, 0, 2)
    qoh3 = qoh.reshape(L, H, NB).transpose(1, 0, 2)
    cnt3 = cnt.reshape(H, NB)[:, None, :]
    tbl_i = tbl.reshape(NROW).astype(jnp.int32)
    kvflat = kv2.reshape(L * H, 2 * DK)

    mesh = plsc.VectorSubcoreMesh(core_axis_name="c", subcore_axis_name="s")
    kvsel = functools.partial(
        pl.kernel,
        mesh=mesh,
        out_type=jax.ShapeDtypeStruct((NROW, 2 * DK), f32),
        scratch_types=[
            pltpu.VMEM((NROW // 32,), jnp.int32),
            pltpu.VMEM((NROW // 32, 2 * DK), f32),
            pltpu.SemaphoreType.DMA,
        ],
    )(_sc_gather)(tbl_i, kvflat)

    kvsel3 = kvsel.reshape(H, NSLOT, 2 * DK)

    out = pl.pallas_call(
        _attn_body,
        grid=(H,),
        in_specs=[
            pl.BlockSpec((1, L, DK), lambda h: (h, 0, 0)),
            pl.BlockSpec((1, NSLOT, 2 * DK), lambda h: (h, 0, 0)),
            pl.BlockSpec((1, DK, RFF), lambda h: (h, 0, 0)),
            pl.BlockSpec((1, 1, RFF), lambda h: (h, 0, 0)),
            pl.BlockSpec((1, DK, RANK), lambda h: (h, 0, 0)),
            pl.BlockSpec((1, RANK, D_MODEL), lambda h: (h, 0, 0)),
            pl.BlockSpec((D_MODEL, D_MODEL), lambda h: (h, 0)),
            pl.BlockSpec((1, 1, NB), lambda h: (h, 0, 0)),
            pl.BlockSpec((1, L, NB), lambda h: (h, 0, 0)),
        ],
        out_specs=pl.BlockSpec((L, D_MODEL), lambda h: (0, 0)),
        out_shape=jax.ShapeDtypeStruct((L, D_MODEL), f32),
    )(q3, kvsel3, omega, rff_bias[:, None, :], U, V, Wo, cnt3, qoh3)

    return (out + bo)[None]
